# Initial kernel scaffold; baseline (speedup 1.0000x reference)
import jax
import jax.numpy as jnp
from jax.experimental import pallas as pl


def kernel(x, edge_index, W_enc, b_enc, W1, b1, g1, beta1, W2, b2, g2, beta2, W_out, b_out):
    raise NotImplementedError("write your pallas kernel here")


# speedup vs baseline: 7.6617x; 7.6617x over previous
import functools

import jax
import jax.numpy as jnp
from jax import lax
from jax.experimental import pallas as pl
from jax.experimental.pallas import tpu as pltpu
from jax.experimental.pallas import tpu_sc as plsc

N = 10000
E = 160000
D = 256
NPAD = 10240
NC = 2
NS = 16
HALF = 5120
TRASH = 5120
ACC_ROWS = 5128
RPT = HALF // NS
EPT = E // NS
K = 80
NB = EPT // K
DW = 16

_MESH = plsc.VectorSubcoreMesh(
    core_axis_name="c", subcore_axis_name="s", num_cores=NC, num_subcores=NS
)


@functools.partial(
    pl.kernel,
    out_type=jax.ShapeDtypeStruct((NPAD, DW), jnp.float32),
    mesh=_MESH,
    scratch_types=[
        pltpu.VMEM_SHARED((ACC_ROWS, DW), jnp.float32),
        pltpu.VMEM((EPT,), jnp.int32),
        pltpu.VMEM((K, DW), jnp.float32),
        pltpu.VMEM((K,), jnp.int32),
    ],
)
def _sc_degree(dst_hbm, ones_hbm, deg_hbm, acc, dst_v, ones_v, sidx):
    c = lax.axis_index("c")
    t = lax.axis_index("s")
    lo = c * HALF
    pltpu.sync_copy(ones_hbm, acc.at[pl.ds(t * RPT, RPT)])
    pltpu.sync_copy(dst_hbm.at[pl.ds(t * EPT, EPT)], dst_v)
    pltpu.sync_copy(ones_hbm.at[pl.ds(0, K)], ones_v)
    plsc.subcore_barrier()

    @pl.loop(0, NB)
    def _batch(b):
        for v in range(K // 16):
            d16 = dst_v[pl.ds(b * K + v * 16, 16)]
            owned = (d16 >= lo) & (d16 < lo + HALF)
            sidx[pl.ds(v * 16, 16)] = jnp.where(owned, d16 - lo, TRASH)
        pltpu.sync_copy(ones_v, acc.at[sidx], add=True)

    plsc.subcore_barrier()
    pltpu.sync_copy(
        acc.at[pl.ds(t * RPT, RPT)],
        deg_hbm.at[pl.ds(c * HALF + t * RPT, RPT)],
    )


NW = NC * NS
EPW = 5040
EPAD = NW * EPW
NBW = EPW // K


@functools.partial(
    pl.kernel,
    out_type=(),
    mesh=_MESH,
    scratch_types=[
        pltpu.VMEM((EPW,), jnp.int32),
        pltpu.VMEM((EPW,), jnp.int32),
        pltpu.VMEM((K,), jnp.int32),
        pltpu.VMEM((K,), jnp.int32),
        pltpu.VMEM((K, D), jnp.float32),
        pltpu.SemaphoreType.DMA,
        pltpu.SemaphoreType.DMA,
    ],
)
def _sc_scatter(src_hbm, dst_hbm, hws_hbm, acc_ref,
                src_v, dst_v, gidx, sidx, rows, gsem, ssem):
    c = lax.axis_index("c")
    t = lax.axis_index("s")
    base = (c * NS + t) * EPW
    pltpu.sync_copy(src_hbm.at[pl.ds(base, EPW)], src_v)
    pltpu.sync_copy(dst_hbm.at[pl.ds(base, EPW)], dst_v)

    @pl.loop(0, NBW)
    def _batch(b):
        for v in range(K // 16):
            gidx[pl.ds(v * 16, 16)] = src_v[pl.ds(b * K + v * 16, 16)]
            sidx[pl.ds(v * 16, 16)] = dst_v[pl.ds(b * K + v * 16, 16)]
        pltpu.async_copy(hws_hbm.at[gidx], rows, gsem).wait()
        pltpu.async_copy(rows, acc_ref.at[sidx], ssem, add=True).wait()


def _ln_relu(acc, hws, dinv_col, b, g, beta):
    t = (acc + hws) * dinv_col + b
    mu = jnp.mean(t, axis=-1, keepdims=True)
    var = jnp.mean((t - mu) ** 2, axis=-1, keepdims=True)
    tn = g * (t - mu) * lax.rsqrt(var + 1e-5) + beta
    return jnp.maximum(tn, 0.0)


def _enc_body(x_ref, deg_ref, wenc_ref, benc_ref, w1_ref,
              h0_ref, dinv_ref, hws1_ref):
    h0 = jnp.maximum(
        jnp.dot(x_ref[...], wenc_ref[...], preferred_element_type=jnp.float32)
        + benc_ref[...], 0.0)
    dinv = lax.rsqrt(deg_ref[...])
    h0_ref[...] = h0
    dinv_ref[...] = dinv
    hws1_ref[...] = jnp.dot(
        h0, w1_ref[...], preferred_element_type=jnp.float32) * dinv[:, 0:1]


def _mid_body(acc_ref, hws_ref, hprev_ref, dinv_ref, b_ref, g_ref, beta_ref,
              w_ref, h_ref, hwsn_ref):
    dinv = dinv_ref[...][:, 0:1]
    h = _ln_relu(acc_ref[...], hws_ref[...], dinv,
                 b_ref[...], g_ref[...], beta_ref[...]) + hprev_ref[...]
    h_ref[...] = h
    hwsn_ref[...] = jnp.dot(
        h, w_ref[...], preferred_element_type=jnp.float32) * dinv


def _fin_body(acc_ref, hws_ref, hprev_ref, dinv_ref, b_ref, g_ref, beta_ref,
              w_ref, bout_ref, out_ref):
    dinv = dinv_ref[...][:, 0:1]
    h = _ln_relu(acc_ref[...], hws_ref[...], dinv,
                 b_ref[...], g_ref[...], beta_ref[...]) + hprev_ref[...]
    out_ref[...] = jnp.dot(
        h, w_ref[...], preferred_element_type=jnp.float32) + bout_ref[...]


_GRID = (NPAD // 320,)
_ROWS = pl.BlockSpec((320, D), lambda i: (i, 0))
_ROWS16 = pl.BlockSpec((320, DW), lambda i: (i, 0))
_WMAT = pl.BlockSpec((D, D), lambda i: (0, 0))
_VECB = pl.BlockSpec((1, D), lambda i: (0, 0))
_F32 = jnp.float32

_enc_call = pl.pallas_call(
    _enc_body,
    grid=_GRID,
    in_specs=[_ROWS, _ROWS16, _WMAT, _VECB, _WMAT],
    out_specs=[_ROWS, _ROWS16, _ROWS],
    out_shape=[
        jax.ShapeDtypeStruct((NPAD, D), _F32),
        jax.ShapeDtypeStruct((NPAD, DW), _F32),
        jax.ShapeDtypeStruct((NPAD, D), _F32),
    ],
)

_mid_call = pl.pallas_call(
    _mid_body,
    grid=_GRID,
    in_specs=[_ROWS, _ROWS, _ROWS, _ROWS16, _VECB, _VECB, _VECB, _WMAT],
    out_specs=[_ROWS, _ROWS],
    out_shape=[
        jax.ShapeDtypeStruct((NPAD, D), _F32),
        jax.ShapeDtypeStruct((NPAD, D), _F32),
    ],
)

_fin_call = pl.pallas_call(
    _fin_body,
    grid=_GRID,
    in_specs=[_ROWS, _ROWS, _ROWS, _ROWS16, _VECB, _VECB, _VECB, _WMAT, _VECB],
    out_specs=_ROWS,
    out_shape=jax.ShapeDtypeStruct((NPAD, D), _F32),
)


def kernel(x, edge_index, W_enc, b_enc, W1, b1, g1, beta1,
           W2, b2, g2, beta2, W_out, b_out):
    src = edge_index[0]
    dst = edge_index[1]
    xp = jnp.concatenate(
        [x, jnp.zeros((NPAD - N, D), jnp.float32)], axis=0)
    ones16 = jnp.ones((RPT, DW), jnp.float32)
    src_p = jnp.concatenate(
        [src, jnp.zeros((EPAD - E,), src.dtype)])
    dst_p = jnp.concatenate(
        [dst, jnp.full((EPAD - E,), N, dst.dtype)])

    deg = _sc_degree(dst, ones16)
    h0, dinv, hws1 = _enc_call(
        xp, deg, W_enc, b_enc.reshape(1, D), W1)
    acc1_ref = jax.new_ref(jnp.zeros((NPAD, D), jnp.float32))
    _sc_scatter(src_p, dst_p, hws1, acc1_ref)
    h1, hws2 = _mid_call(
        acc1_ref[...], hws1, h0, dinv, b1.reshape(1, D), g1.reshape(1, D),
        beta1.reshape(1, D), W2)
    acc2_ref = jax.new_ref(jnp.zeros((NPAD, D), jnp.float32))
    _sc_scatter(src_p, dst_p, hws2, acc2_ref)
    out = _fin_call(
        acc2_ref[...], hws2, h1, dinv, b2.reshape(1, D), g2.reshape(1, D),
        beta2.reshape(1, D), W_out, b_out.reshape(1, D))
    return out[:N]


# speedup vs baseline: 12.8597x; 1.6784x over previous
import functools

import jax
import jax.numpy as jnp
from jax import lax
from jax.experimental import pallas as pl
from jax.experimental.pallas import tpu as pltpu
from jax.experimental.pallas import tpu_sc as plsc

N = 10000
E = 160000
D = 256
NPAD = 10240
NC = 2
NS = 16
NW = NC * NS
K = 128
EPW = 5120
EPAD = NW * EPW
NBW = EPW // K
DW = 16

_MESH = plsc.VectorSubcoreMesh(
    core_axis_name="c", subcore_axis_name="s", num_cores=NC, num_subcores=NS
)


HALF = 5120
TRASH = 5120
ACC_ROWS = 5128
RPT = HALF // NS


@functools.partial(
    pl.kernel,
    out_type=jax.ShapeDtypeStruct((NPAD, DW), jnp.float32),
    mesh=_MESH,
    scratch_types=[
        pltpu.VMEM_SHARED((ACC_ROWS, DW), jnp.float32),
        pltpu.VMEM((EPW,), jnp.int32),
        pltpu.VMEM((K, DW), jnp.float32),
        pltpu.VMEM((K,), jnp.int32),
        pltpu.VMEM((K,), jnp.int32),
        pltpu.SemaphoreType.DMA,
        pltpu.SemaphoreType.DMA,
    ],
)
def _sc_degree(dst_hbm, ones_hbm, deg_hbm,
               acc, dst_v, ones_v, sidx0, sidx1, sem0, sem1):
    sidx = (sidx0, sidx1)
    sem = (sem0, sem1)
    c = lax.axis_index("c")
    t = lax.axis_index("s")
    lo = c * HALF
    pltpu.sync_copy(ones_hbm, acc.at[pl.ds(t * RPT, RPT)])
    pltpu.sync_copy(dst_hbm.at[pl.ds((c * NS + t) * EPW, EPW)], dst_v)
    pltpu.sync_copy(ones_hbm.at[pl.ds(0, K)], ones_v)
    plsc.subcore_barrier()

    def build(b, s):
        for v in range(K // 16):
            d16 = dst_v[pl.ds(b * K + v * 16, 16)]
            owned = (d16 >= lo) & (d16 < lo + HALF)
            sidx[s][pl.ds(v * 16, 16)] = jnp.where(owned, d16 - lo, TRASH)

    def sstart(s):
        pltpu.async_copy(ones_v, acc.at[sidx[s]], sem[s], add=True)

    def swait(s):
        pltpu.make_async_copy(ones_v, acc.at[sidx[s]], sem[s]).wait()

    for s in range(2):
        build(s, s)
        sstart(s)

    @pl.loop(0, NBW // 2 - 1)
    def _pair(i):
        for s in range(2):
            swait(s)
            build(2 * i + 2 + s, s)
            sstart(s)

    swait(0)
    swait(1)
    plsc.subcore_barrier()
    pltpu.sync_copy(
        acc.at[pl.ds(t * RPT, RPT)],
        deg_hbm.at[pl.ds(c * HALF + t * RPT, RPT)],
    )


@functools.partial(
    pl.kernel,
    out_type=(),
    mesh=_MESH,
    scratch_types=[
        pltpu.VMEM((EPW,), jnp.int32),
        pltpu.VMEM((EPW,), jnp.int32),
        pltpu.VMEM((K,), jnp.int32),
        pltpu.VMEM((K,), jnp.int32),
        pltpu.VMEM((K,), jnp.int32),
        pltpu.VMEM((K,), jnp.int32),
        pltpu.VMEM((K, D), jnp.float32),
        pltpu.VMEM((K, D), jnp.float32),
        pltpu.SemaphoreType.DMA,
        pltpu.SemaphoreType.DMA,
        pltpu.SemaphoreType.DMA,
        pltpu.SemaphoreType.DMA,
    ],
)
def _sc_scatter(src_hbm, dst_hbm, hws_hbm, acc_ref,
                src_v, dst_v, gidx0, gidx1, sidx0, sidx1,
                rows0, rows1, gsem0, gsem1, ssem0, ssem1):
    gidx = (gidx0, gidx1)
    sidx = (sidx0, sidx1)
    rows = (rows0, rows1)
    gsem = (gsem0, gsem1)
    ssem = (ssem0, ssem1)
    base = (lax.axis_index("c") * NS + lax.axis_index("s")) * EPW
    pltpu.sync_copy(src_hbm.at[pl.ds(base, EPW)], src_v)
    pltpu.sync_copy(dst_hbm.at[pl.ds(base, EPW)], dst_v)

    def gstart(b, s):
        for v in range(K // 16):
            gidx[s][pl.ds(v * 16, 16)] = src_v[pl.ds(b * K + v * 16, 16)]
            sidx[s][pl.ds(v * 16, 16)] = dst_v[pl.ds(b * K + v * 16, 16)]
        pltpu.async_copy(hws_hbm.at[gidx[s]], rows[s], gsem[s])

    def gwait(s):
        pltpu.make_async_copy(hws_hbm.at[gidx[s]], rows[s], gsem[s]).wait()

    def sstart(s):
        pltpu.async_copy(rows[s], acc_ref.at[sidx[s]], ssem[s], add=True)

    def swait(s):
        pltpu.make_async_copy(rows[s], acc_ref.at[sidx[s]], ssem[s]).wait()

    for s in range(2):
        gstart(s, s)

    @pl.loop(0, NBW // 2 - 1)
    def _pair(i):
        for s in range(2):
            gwait(s)
            sstart(s)
            swait(s)
            gstart(2 * i + 2 + s, s)

    for s in range(2):
        gwait(s)
        sstart(s)
        swait(s)


def _ln_relu(acc, hws, dinv_col, b, g, beta):
    t = (acc + hws) * dinv_col + b
    mu = jnp.mean(t, axis=-1, keepdims=True)
    var = jnp.mean((t - mu) ** 2, axis=-1, keepdims=True)
    tn = g * (t - mu) * lax.rsqrt(var + 1e-5) + beta
    return jnp.maximum(tn, 0.0)


def _enc_body(x_ref, deg_ref, wenc_ref, benc_ref, w1_ref,
              h0_ref, dinv_ref, hws1_ref):
    h0 = jnp.maximum(
        jnp.dot(x_ref[...], wenc_ref[...], preferred_element_type=jnp.float32)
        + benc_ref[...], 0.0)
    dinv = lax.rsqrt(deg_ref[...])
    h0_ref[...] = h0
    dinv_ref[...] = dinv
    hws1_ref[...] = jnp.dot(
        h0, w1_ref[...], preferred_element_type=jnp.float32) * dinv[:, 0:1]


def _mid_body(acc_ref, hws_ref, hprev_ref, dinv_ref, b_ref, g_ref, beta_ref,
              w_ref, h_ref, hwsn_ref):
    dinv = dinv_ref[...][:, 0:1]
    h = _ln_relu(acc_ref[...], hws_ref[...], dinv,
                 b_ref[...], g_ref[...], beta_ref[...]) + hprev_ref[...]
    h_ref[...] = h
    hwsn_ref[...] = jnp.dot(
        h, w_ref[...], preferred_element_type=jnp.float32) * dinv


def _fin_body(acc_ref, hws_ref, hprev_ref, dinv_ref, b_ref, g_ref, beta_ref,
              w_ref, bout_ref, out_ref):
    dinv = dinv_ref[...][:, 0:1]
    h = _ln_relu(acc_ref[...], hws_ref[...], dinv,
                 b_ref[...], g_ref[...], beta_ref[...]) + hprev_ref[...]
    out_ref[...] = jnp.dot(
        h, w_ref[...], preferred_element_type=jnp.float32) + bout_ref[...]


_GRID = (NPAD // 320,)
_ROWS = pl.BlockSpec((320, D), lambda i: (i, 0))
_ROWS16 = pl.BlockSpec((320, DW), lambda i: (i, 0))
_WMAT = pl.BlockSpec((D, D), lambda i: (0, 0))
_VECB = pl.BlockSpec((1, D), lambda i: (0, 0))
_F32 = jnp.float32

_enc_call = pl.pallas_call(
    _enc_body,
    grid=_GRID,
    in_specs=[_ROWS, _ROWS16, _WMAT, _VECB, _WMAT],
    out_specs=[_ROWS, _ROWS16, _ROWS],
    out_shape=[
        jax.ShapeDtypeStruct((NPAD, D), _F32),
        jax.ShapeDtypeStruct((NPAD, DW), _F32),
        jax.ShapeDtypeStruct((NPAD, D), _F32),
    ],
)

_mid_call = pl.pallas_call(
    _mid_body,
    grid=_GRID,
    in_specs=[_ROWS, _ROWS, _ROWS, _ROWS16, _VECB, _VECB, _VECB, _WMAT],
    out_specs=[_ROWS, _ROWS],
    out_shape=[
        jax.ShapeDtypeStruct((NPAD, D), _F32),
        jax.ShapeDtypeStruct((NPAD, D), _F32),
    ],
)

_fin_call = pl.pallas_call(
    _fin_body,
    grid=_GRID,
    in_specs=[_ROWS, _ROWS, _ROWS, _ROWS16, _VECB, _VECB, _VECB, _WMAT, _VECB],
    out_specs=_ROWS,
    out_shape=jax.ShapeDtypeStruct((NPAD, D), _F32),
)


def kernel(x, edge_index, W_enc, b_enc, W1, b1, g1, beta1,
           W2, b2, g2, beta2, W_out, b_out):
    src = edge_index[0]
    dst = edge_index[1]
    xp = jnp.concatenate(
        [x, jnp.zeros((NPAD - N, D), jnp.float32)], axis=0)
    pad = jnp.arange(EPAD - E, dtype=src.dtype)
    src_p = jnp.concatenate([src, pad % 256])
    dst_p = jnp.concatenate([dst, N + pad % (NPAD - N)])
    ones_k = jnp.ones((RPT, DW), jnp.float32)

    deg = _sc_degree(dst_p, ones_k)
    h0, dinv, hws1 = _enc_call(
        xp, deg, W_enc, b_enc.reshape(1, D), W1)
    acc1_ref = jax.new_ref(jnp.zeros((NPAD, D), jnp.float32))
    _sc_scatter(src_p, dst_p, hws1, acc1_ref)
    h1, hws2 = _mid_call(
        acc1_ref[...], hws1, h0, dinv, b1.reshape(1, D), g1.reshape(1, D),
        beta1.reshape(1, D), W2)
    acc2_ref = jax.new_ref(jnp.zeros((NPAD, D), jnp.float32))
    _sc_scatter(src_p, dst_p, hws2, acc2_ref)
    out = _fin_call(
        acc2_ref[...], hws2, h1, dinv, b2.reshape(1, D), g2.reshape(1, D),
        beta2.reshape(1, D), W_out, b_out.reshape(1, D))
    return out[:N]


# speedup vs baseline: 12.9471x; 1.0068x over previous
import functools

import jax
import jax.numpy as jnp
from jax import lax
from jax.experimental import pallas as pl
from jax.experimental.pallas import tpu as pltpu
from jax.experimental.pallas import tpu_sc as plsc

N = 10000
E = 160000
D = 256
NPAD = 10240
NC = 2
NS = 16
NW = NC * NS
K = 112
EPW = 5040
EPAD = NW * EPW
NBW = EPW // K
NSLOT = 3
DW = 16

_MESH = plsc.VectorSubcoreMesh(
    core_axis_name="c", subcore_axis_name="s", num_cores=NC, num_subcores=NS
)


HALF = 5120
TRASH = 5120
ACC_ROWS = 5128
RPT = HALF // NS


@functools.partial(
    pl.kernel,
    out_type=jax.ShapeDtypeStruct((NPAD, DW), jnp.float32),
    mesh=_MESH,
    scratch_types=[
        pltpu.VMEM_SHARED((ACC_ROWS, DW), jnp.float32),
        pltpu.VMEM((EPW,), jnp.int32),
        pltpu.VMEM((K, DW), jnp.float32),
        pltpu.VMEM((K,), jnp.int32),
        pltpu.VMEM((K,), jnp.int32),
        pltpu.VMEM((K,), jnp.int32),
        pltpu.SemaphoreType.DMA,
        pltpu.SemaphoreType.DMA,
        pltpu.SemaphoreType.DMA,
    ],
)
def _sc_degree(dst_hbm, ones_hbm, deg_hbm,
               acc, dst_v, ones_v, sidx0, sidx1, sidx2, sem0, sem1, sem2):
    sidx = (sidx0, sidx1, sidx2)
    sem = (sem0, sem1, sem2)
    c = lax.axis_index("c")
    t = lax.axis_index("s")
    lo = c * HALF
    pltpu.sync_copy(ones_hbm, acc.at[pl.ds(t * RPT, RPT)])
    pltpu.sync_copy(dst_hbm.at[pl.ds((c * NS + t) * EPW, EPW)], dst_v)
    pltpu.sync_copy(ones_hbm.at[pl.ds(0, K)], ones_v)
    plsc.subcore_barrier()

    def build(b, s):
        for v in range(K // 16):
            d16 = dst_v[pl.ds(b * K + v * 16, 16)]
            owned = (d16 >= lo) & (d16 < lo + HALF)
            sidx[s][pl.ds(v * 16, 16)] = jnp.where(owned, d16 - lo, TRASH)

    def sstart(s):
        pltpu.async_copy(ones_v, acc.at[sidx[s]], sem[s], add=True)

    def swait(s):
        pltpu.make_async_copy(ones_v, acc.at[sidx[s]], sem[s]).wait()

    for s in range(NSLOT):
        build(s, s)
        sstart(s)

    @pl.loop(0, NBW // NSLOT - 1)
    def _round(i):
        for s in range(NSLOT):
            swait(s)
            build(NSLOT * i + NSLOT + s, s)
            sstart(s)

    for s in range(NSLOT):
        swait(s)
    plsc.subcore_barrier()
    pltpu.sync_copy(
        acc.at[pl.ds(t * RPT, RPT)],
        deg_hbm.at[pl.ds(c * HALF + t * RPT, RPT)],
    )


@functools.partial(
    pl.kernel,
    out_type=(),
    mesh=_MESH,
    scratch_types=[
        pltpu.VMEM((EPW,), jnp.int32),
        pltpu.VMEM((EPW,), jnp.int32),
        pltpu.VMEM((K,), jnp.int32),
        pltpu.VMEM((K,), jnp.int32),
        pltpu.VMEM((K,), jnp.int32),
        pltpu.VMEM((K,), jnp.int32),
        pltpu.VMEM((K,), jnp.int32),
        pltpu.VMEM((K,), jnp.int32),
        pltpu.VMEM((K, D), jnp.float32),
        pltpu.VMEM((K, D), jnp.float32),
        pltpu.VMEM((K, D), jnp.float32),
        pltpu.SemaphoreType.DMA,
        pltpu.SemaphoreType.DMA,
        pltpu.SemaphoreType.DMA,
        pltpu.SemaphoreType.DMA,
        pltpu.SemaphoreType.DMA,
        pltpu.SemaphoreType.DMA,
    ],
)
def _sc_scatter(src_hbm, dst_hbm, hws_hbm, acc_ref,
                src_v, dst_v, gidx0, gidx1, gidx2, sidx0, sidx1, sidx2,
                rows0, rows1, rows2, gsem0, gsem1, gsem2,
                ssem0, ssem1, ssem2):
    gidx = (gidx0, gidx1, gidx2)
    sidx = (sidx0, sidx1, sidx2)
    rows = (rows0, rows1, rows2)
    gsem = (gsem0, gsem1, gsem2)
    ssem = (ssem0, ssem1, ssem2)
    base = (lax.axis_index("c") * NS + lax.axis_index("s")) * EPW
    pltpu.sync_copy(src_hbm.at[pl.ds(base, EPW)], src_v)
    pltpu.sync_copy(dst_hbm.at[pl.ds(base, EPW)], dst_v)

    def gstart(b, s):
        for v in range(K // 16):
            gidx[s][pl.ds(v * 16, 16)] = src_v[pl.ds(b * K + v * 16, 16)]
            sidx[s][pl.ds(v * 16, 16)] = dst_v[pl.ds(b * K + v * 16, 16)]
        pltpu.async_copy(hws_hbm.at[gidx[s]], rows[s], gsem[s])

    def gwait(s):
        pltpu.make_async_copy(hws_hbm.at[gidx[s]], rows[s], gsem[s]).wait()

    def sstart(s):
        pltpu.async_copy(rows[s], acc_ref.at[sidx[s]], ssem[s], add=True)

    def swait(s):
        pltpu.make_async_copy(rows[s], acc_ref.at[sidx[s]], ssem[s]).wait()

    for s in range(NSLOT):
        gstart(s, s)

    @pl.loop(0, NBW // NSLOT - 1)
    def _round(i):
        for s in range(NSLOT):
            gwait(s)
            sstart(s)
        for s in range(NSLOT):
            swait(s)
            gstart(NSLOT * i + NSLOT + s, s)

    for s in range(NSLOT):
        gwait(s)
        sstart(s)
    for s in range(NSLOT):
        swait(s)


def _ln_relu(acc, hws, dinv_col, b, g, beta):
    t = (acc + hws) * dinv_col + b
    mu = jnp.mean(t, axis=-1, keepdims=True)
    var = jnp.mean((t - mu) ** 2, axis=-1, keepdims=True)
    tn = g * (t - mu) * lax.rsqrt(var + 1e-5) + beta
    return jnp.maximum(tn, 0.0)


def _enc_body(x_ref, deg_ref, wenc_ref, benc_ref, w1_ref,
              h0_ref, dinv_ref, hws1_ref):
    h0 = jnp.maximum(
        jnp.dot(x_ref[...], wenc_ref[...], preferred_element_type=jnp.float32)
        + benc_ref[...], 0.0)
    dinv = lax.rsqrt(deg_ref[...])
    h0_ref[...] = h0
    dinv_ref[...] = dinv
    hws1_ref[...] = jnp.dot(
        h0, w1_ref[...], preferred_element_type=jnp.float32) * dinv[:, 0:1]


def _mid_body(acc_ref, hws_ref, hprev_ref, dinv_ref, b_ref, g_ref, beta_ref,
              w_ref, h_ref, hwsn_ref):
    dinv = dinv_ref[...][:, 0:1]
    h = _ln_relu(acc_ref[...], hws_ref[...], dinv,
                 b_ref[...], g_ref[...], beta_ref[...]) + hprev_ref[...]
    h_ref[...] = h
    hwsn_ref[...] = jnp.dot(
        h, w_ref[...], preferred_element_type=jnp.float32) * dinv


def _fin_body(acc_ref, hws_ref, hprev_ref, dinv_ref, b_ref, g_ref, beta_ref,
              w_ref, bout_ref, out_ref):
    dinv = dinv_ref[...][:, 0:1]
    h = _ln_relu(acc_ref[...], hws_ref[...], dinv,
                 b_ref[...], g_ref[...], beta_ref[...]) + hprev_ref[...]
    out_ref[...] = jnp.dot(
        h, w_ref[...], preferred_element_type=jnp.float32) + bout_ref[...]


_GRID = (NPAD // 320,)
_ROWS = pl.BlockSpec((320, D), lambda i: (i, 0))
_ROWS16 = pl.BlockSpec((320, DW), lambda i: (i, 0))
_WMAT = pl.BlockSpec((D, D), lambda i: (0, 0))
_VECB = pl.BlockSpec((1, D), lambda i: (0, 0))
_F32 = jnp.float32

_enc_call = pl.pallas_call(
    _enc_body,
    grid=_GRID,
    in_specs=[_ROWS, _ROWS16, _WMAT, _VECB, _WMAT],
    out_specs=[_ROWS, _ROWS16, _ROWS],
    out_shape=[
        jax.ShapeDtypeStruct((NPAD, D), _F32),
        jax.ShapeDtypeStruct((NPAD, DW), _F32),
        jax.ShapeDtypeStruct((NPAD, D), _F32),
    ],
)

_mid_call = pl.pallas_call(
    _mid_body,
    grid=_GRID,
    in_specs=[_ROWS, _ROWS, _ROWS, _ROWS16, _VECB, _VECB, _VECB, _WMAT],
    out_specs=[_ROWS, _ROWS],
    out_shape=[
        jax.ShapeDtypeStruct((NPAD, D), _F32),
        jax.ShapeDtypeStruct((NPAD, D), _F32),
    ],
)

_fin_call = pl.pallas_call(
    _fin_body,
    grid=_GRID,
    in_specs=[_ROWS, _ROWS, _ROWS, _ROWS16, _VECB, _VECB, _VECB, _WMAT, _VECB],
    out_specs=_ROWS,
    out_shape=jax.ShapeDtypeStruct((NPAD, D), _F32),
)


def kernel(x, edge_index, W_enc, b_enc, W1, b1, g1, beta1,
           W2, b2, g2, beta2, W_out, b_out):
    src = edge_index[0]
    dst = edge_index[1]
    xp = jnp.concatenate(
        [x, jnp.zeros((NPAD - N, D), jnp.float32)], axis=0)
    pad = jnp.arange(EPAD - E, dtype=src.dtype)
    src_p = jnp.concatenate([src, pad % 256])
    dst_p = jnp.concatenate([dst, N + pad % (NPAD - N)])
    ones_k = jnp.ones((RPT, DW), jnp.float32)

    deg = _sc_degree(dst_p, ones_k)
    h0, dinv, hws1 = _enc_call(
        xp, deg, W_enc, b_enc.reshape(1, D), W1)
    acc1_ref = jax.new_ref(jnp.zeros((NPAD, D), jnp.float32))
    _sc_scatter(src_p, dst_p, hws1, acc1_ref)
    h1, hws2 = _mid_call(
        acc1_ref[...], hws1, h0, dinv, b1.reshape(1, D), g1.reshape(1, D),
        beta1.reshape(1, D), W2)
    acc2_ref = jax.new_ref(jnp.zeros((NPAD, D), jnp.float32))
    _sc_scatter(src_p, dst_p, hws2, acc2_ref)
    out = _fin_call(
        acc2_ref[...], hws2, h1, dinv, b2.reshape(1, D), g2.reshape(1, D),
        beta2.reshape(1, D), W_out, b_out.reshape(1, D))
    return out[:N]


# speedup vs baseline: 13.2631x; 1.0244x over previous
import functools

import jax
import jax.numpy as jnp
from jax import lax
from jax.experimental import pallas as pl
from jax.experimental.pallas import tpu as pltpu
from jax.experimental.pallas import tpu_sc as plsc

N = 10000
E = 160000
D = 256
NPAD = 10240
NC = 2
NS = 16
NW = NC * NS
K = 112
EPW = 5040
EPAD = NW * EPW
NBW = EPW // K
NSLOT = 3
DW = 16

_MESH = plsc.VectorSubcoreMesh(
    core_axis_name="c", subcore_axis_name="s", num_cores=NC, num_subcores=NS
)


HALF = 5120
TRASH = 5120
ACC_ROWS = 5128
RPT = HALF // NS


@functools.partial(
    pl.kernel,
    out_type=jax.ShapeDtypeStruct((NPAD, DW), jnp.float32),
    mesh=_MESH,
    scratch_types=[
        pltpu.VMEM_SHARED((ACC_ROWS, DW), jnp.float32),
        pltpu.VMEM((EPW,), jnp.int32),
        pltpu.VMEM((K, DW), jnp.float32),
        pltpu.VMEM((K,), jnp.int32),
        pltpu.VMEM((K,), jnp.int32),
        pltpu.VMEM((K,), jnp.int32),
        pltpu.SemaphoreType.DMA,
        pltpu.SemaphoreType.DMA,
        pltpu.SemaphoreType.DMA,
    ],
)
def _sc_degree(dst_hbm, ones_hbm, deg_hbm,
               acc, dst_v, ones_v, sidx0, sidx1, sidx2, sem0, sem1, sem2):
    sidx = (sidx0, sidx1, sidx2)
    sem = (sem0, sem1, sem2)
    c = lax.axis_index("c")
    t = lax.axis_index("s")
    lo = c * HALF
    pltpu.sync_copy(ones_hbm, acc.at[pl.ds(t * RPT, RPT)])
    pltpu.sync_copy(dst_hbm.at[pl.ds((c * NS + t) * EPW, EPW)], dst_v)
    pltpu.sync_copy(ones_hbm.at[pl.ds(0, K)], ones_v)
    plsc.subcore_barrier()

    def build(b, s):
        for v in range(K // 16):
            d16 = dst_v[pl.ds(b * K + v * 16, 16)]
            owned = (d16 >= lo) & (d16 < lo + HALF)
            sidx[s][pl.ds(v * 16, 16)] = jnp.where(owned, d16 - lo, TRASH)

    def sstart(s):
        pltpu.async_copy(ones_v, acc.at[sidx[s]], sem[s], add=True)

    def swait(s):
        pltpu.make_async_copy(ones_v, acc.at[sidx[s]], sem[s]).wait()

    for s in range(NSLOT):
        build(s, s)
        sstart(s)

    @pl.loop(0, NBW // NSLOT - 1)
    def _round(i):
        for s in range(NSLOT):
            swait(s)
            build(NSLOT * i + NSLOT + s, s)
            sstart(s)

    for s in range(NSLOT):
        swait(s)
    plsc.subcore_barrier()
    pltpu.sync_copy(
        acc.at[pl.ds(t * RPT, RPT)],
        deg_hbm.at[pl.ds(c * HALF + t * RPT, RPT)],
    )


@functools.partial(
    pl.kernel,
    out_type=(),
    mesh=_MESH,
    scratch_types=[
        pltpu.VMEM((EPW,), jnp.int32),
        pltpu.VMEM((EPW,), jnp.int32),
        pltpu.VMEM((K,), jnp.int32),
        pltpu.VMEM((K,), jnp.int32),
        pltpu.VMEM((K,), jnp.int32),
        pltpu.VMEM((K,), jnp.int32),
        pltpu.VMEM((K,), jnp.int32),
        pltpu.VMEM((K,), jnp.int32),
        pltpu.VMEM((K, D), jnp.float32),
        pltpu.VMEM((K, D), jnp.float32),
        pltpu.VMEM((K, D), jnp.float32),
        pltpu.SemaphoreType.DMA,
        pltpu.SemaphoreType.DMA,
        pltpu.SemaphoreType.DMA,
        pltpu.SemaphoreType.DMA,
        pltpu.SemaphoreType.DMA,
        pltpu.SemaphoreType.DMA,
    ],
)
def _sc_scatter(src_hbm, dst_hbm, hws_hbm, acc_ref,
                src_v, dst_v, gidx0, gidx1, gidx2, sidx0, sidx1, sidx2,
                rows0, rows1, rows2, gsem0, gsem1, gsem2,
                ssem0, ssem1, ssem2):
    gidx = (gidx0, gidx1, gidx2)
    sidx = (sidx0, sidx1, sidx2)
    rows = (rows0, rows1, rows2)
    gsem = (gsem0, gsem1, gsem2)
    ssem = (ssem0, ssem1, ssem2)
    base = (lax.axis_index("c") * NS + lax.axis_index("s")) * EPW
    pltpu.sync_copy(src_hbm.at[pl.ds(base, EPW)], src_v)
    pltpu.sync_copy(dst_hbm.at[pl.ds(base, EPW)], dst_v)

    def gstart(b, s):
        for v in range(K // 16):
            gidx[s][pl.ds(v * 16, 16)] = src_v[pl.ds(b * K + v * 16, 16)]
            sidx[s][pl.ds(v * 16, 16)] = dst_v[pl.ds(b * K + v * 16, 16)]
        pltpu.async_copy(hws_hbm.at[gidx[s]], rows[s], gsem[s])

    def gwait(s):
        pltpu.make_async_copy(hws_hbm.at[gidx[s]], rows[s], gsem[s]).wait()

    def sstart(s):
        pltpu.async_copy(rows[s], acc_ref.at[sidx[s]], ssem[s], add=True)

    def swait(s):
        pltpu.make_async_copy(rows[s], acc_ref.at[sidx[s]], ssem[s]).wait()

    for s in range(NSLOT):
        gstart(s, s)

    @pl.loop(0, NBW // NSLOT - 1)
    def _round(i):
        for s in range(NSLOT):
            gwait(s)
            sstart(s)
        for s in range(NSLOT):
            swait(s)
            gstart(NSLOT * i + NSLOT + s, s)

    for s in range(NSLOT):
        gwait(s)
        sstart(s)
    for s in range(NSLOT):
        swait(s)


def _ln_relu(acc, hws, dinv_col, b, g, beta):
    t = (acc + hws) * dinv_col + b
    mu = jnp.mean(t, axis=-1, keepdims=True)
    var = jnp.mean((t - mu) ** 2, axis=-1, keepdims=True)
    tn = g * (t - mu) * lax.rsqrt(var + 1e-5) + beta
    return jnp.maximum(tn, 0.0)


def _enc_body(x_ref, wenc_ref, benc_ref, w1_ref, h0_ref, hw1_ref):
    h0 = jnp.maximum(
        jnp.dot(x_ref[...], wenc_ref[...], preferred_element_type=jnp.float32)
        + benc_ref[...], 0.0)
    h0_ref[...] = h0
    hw1_ref[...] = jnp.dot(
        h0, w1_ref[...], preferred_element_type=jnp.float32)


def _scale_body(deg_ref, hw1_ref, dinv_ref, hws1_ref):
    dinv = lax.rsqrt(deg_ref[...])
    dinv_ref[...] = dinv
    hws1_ref[...] = hw1_ref[...] * dinv[:, 0:1]


def _mid_body(acc_ref, hws_ref, hprev_ref, dinv_ref, b_ref, g_ref, beta_ref,
              w_ref, h_ref, hwsn_ref):
    dinv = dinv_ref[...][:, 0:1]
    h = _ln_relu(acc_ref[...], hws_ref[...], dinv,
                 b_ref[...], g_ref[...], beta_ref[...]) + hprev_ref[...]
    h_ref[...] = h
    hwsn_ref[...] = jnp.dot(
        h, w_ref[...], preferred_element_type=jnp.float32) * dinv


def _fin_body(acc_ref, hws_ref, hprev_ref, dinv_ref, b_ref, g_ref, beta_ref,
              w_ref, bout_ref, out_ref):
    dinv = dinv_ref[...][:, 0:1]
    h = _ln_relu(acc_ref[...], hws_ref[...], dinv,
                 b_ref[...], g_ref[...], beta_ref[...]) + hprev_ref[...]
    out_ref[...] = jnp.dot(
        h, w_ref[...], preferred_element_type=jnp.float32) + bout_ref[...]


_GRID = (NPAD // 320,)
_ROWS = pl.BlockSpec((320, D), lambda i: (i, 0))
_ROWS16 = pl.BlockSpec((320, DW), lambda i: (i, 0))
_WMAT = pl.BlockSpec((D, D), lambda i: (0, 0))
_VECB = pl.BlockSpec((1, D), lambda i: (0, 0))
_F32 = jnp.float32

_enc_call = pl.pallas_call(
    _enc_body,
    grid=_GRID,
    in_specs=[_ROWS, _WMAT, _VECB, _WMAT],
    out_specs=[_ROWS, _ROWS],
    out_shape=[
        jax.ShapeDtypeStruct((NPAD, D), _F32),
        jax.ShapeDtypeStruct((NPAD, D), _F32),
    ],
)

_scale_call = pl.pallas_call(
    _scale_body,
    grid=_GRID,
    in_specs=[_ROWS16, _ROWS],
    out_specs=[_ROWS16, _ROWS],
    out_shape=[
        jax.ShapeDtypeStruct((NPAD, DW), _F32),
        jax.ShapeDtypeStruct((NPAD, D), _F32),
    ],
)

_mid_call = pl.pallas_call(
    _mid_body,
    grid=_GRID,
    in_specs=[_ROWS, _ROWS, _ROWS, _ROWS16, _VECB, _VECB, _VECB, _WMAT],
    out_specs=[_ROWS, _ROWS],
    out_shape=[
        jax.ShapeDtypeStruct((NPAD, D), _F32),
        jax.ShapeDtypeStruct((NPAD, D), _F32),
    ],
)

_fin_call = pl.pallas_call(
    _fin_body,
    grid=_GRID,
    in_specs=[_ROWS, _ROWS, _ROWS, _ROWS16, _VECB, _VECB, _VECB, _WMAT, _VECB],
    out_specs=_ROWS,
    out_shape=jax.ShapeDtypeStruct((NPAD, D), _F32),
)


def kernel(x, edge_index, W_enc, b_enc, W1, b1, g1, beta1,
           W2, b2, g2, beta2, W_out, b_out):
    src = edge_index[0]
    dst = edge_index[1]
    xp = jnp.concatenate(
        [x, jnp.zeros((NPAD - N, D), jnp.float32)], axis=0)
    pad = jnp.arange(EPAD - E, dtype=src.dtype)
    src_p = jnp.concatenate([src, pad % 256])
    dst_p = jnp.concatenate([dst, N + pad % (NPAD - N)])
    ones_k = jnp.ones((RPT, DW), jnp.float32)

    deg = _sc_degree(dst_p, ones_k)
    h0, hw1 = _enc_call(xp, W_enc, b_enc.reshape(1, D), W1)
    dinv, hws1 = _scale_call(deg, hw1)
    acc1_ref = jax.new_ref(jnp.zeros((NPAD, D), jnp.float32))
    _sc_scatter(src_p, dst_p, hws1, acc1_ref)
    h1, hws2 = _mid_call(
        acc1_ref[...], hws1, h0, dinv, b1.reshape(1, D), g1.reshape(1, D),
        beta1.reshape(1, D), W2)
    acc2_ref = jax.new_ref(jnp.zeros((NPAD, D), jnp.float32))
    _sc_scatter(src_p, dst_p, hws2, acc2_ref)
    out = _fin_call(
        acc2_ref[...], hws2, h1, dinv, b2.reshape(1, D), g2.reshape(1, D),
        beta2.reshape(1, D), W_out, b_out.reshape(1, D))
    return out[:N]


# speedup vs baseline: 13.7885x; 1.0396x over previous
import functools

import jax
import jax.numpy as jnp
from jax import lax
from jax.experimental import pallas as pl
from jax.experimental.pallas import tpu as pltpu
from jax.experimental.pallas import tpu_sc as plsc

N = 10000
E = 160000
D = 256
NPAD = 10240
NC = 2
NS = 16
NW = NC * NS
K = 112
EPW = 5040
EPAD = NW * EPW
NBW = EPW // K
NSLOT = 3
DW = 16

_MESH = plsc.VectorSubcoreMesh(
    core_axis_name="c", subcore_axis_name="s", num_cores=NC, num_subcores=NS
)


HALF = 5120
TRASH = 5120
ACC_ROWS = 5128
RPT = HALF // NS


def _stage_edges(ei_hbm, sect, buf, w, pad_fn):
    base = sect + w * EPW
    tail = E - (NW - 1) * EPW
    npad = EPW - tail

    @pl.when(w < NW - 1)
    def _():
        pltpu.sync_copy(ei_hbm.at[pl.ds(base, EPW)], buf)

    @pl.when(w == NW - 1)
    def _():
        pltpu.sync_copy(ei_hbm.at[pl.ds(sect + (NW - 1) * EPW, tail)],
                        buf.at[pl.ds(0, tail)])
        lanes = lax.iota(jnp.int32, 16)
        for j in range(npad // 16):
            buf[pl.ds(tail + j * 16, 16)] = pad_fn(lanes, j)


def _pad_src(lanes, j):
    return lanes * 16 + (j % 16)


def _pad_dst(lanes, j):
    return N + lanes * 15 + (j % 15)


@functools.partial(
    pl.kernel,
    out_type=jax.ShapeDtypeStruct((NPAD, DW), jnp.float32),
    mesh=_MESH,
    scratch_types=[
        pltpu.VMEM_SHARED((ACC_ROWS, DW), jnp.float32),
        pltpu.VMEM((EPW,), jnp.int32),
        pltpu.VMEM((K, DW), jnp.float32),
        pltpu.VMEM((K,), jnp.int32),
        pltpu.VMEM((K,), jnp.int32),
        pltpu.VMEM((K,), jnp.int32),
        pltpu.SemaphoreType.DMA,
        pltpu.SemaphoreType.DMA,
        pltpu.SemaphoreType.DMA,
    ],
)
def _sc_degree(dst_hbm, ones_hbm, deg_hbm,
               acc, dst_v, ones_v, sidx0, sidx1, sidx2, sem0, sem1, sem2):
    sidx = (sidx0, sidx1, sidx2)
    sem = (sem0, sem1, sem2)
    c = lax.axis_index("c")
    t = lax.axis_index("s")
    lo = c * HALF
    pltpu.sync_copy(ones_hbm, acc.at[pl.ds(t * RPT, RPT)])
    _stage_edges(dst_hbm, E, dst_v, c * NS + t, _pad_dst)
    pltpu.sync_copy(ones_hbm.at[pl.ds(0, K)], ones_v)
    plsc.subcore_barrier()

    def build(b, s):
        for v in range(K // 16):
            d16 = dst_v[pl.ds(b * K + v * 16, 16)]
            owned = (d16 >= lo) & (d16 < lo + HALF)
            sidx[s][pl.ds(v * 16, 16)] = jnp.where(owned, d16 - lo, TRASH)

    def sstart(s):
        pltpu.async_copy(ones_v, acc.at[sidx[s]], sem[s], add=True)

    def swait(s):
        pltpu.make_async_copy(ones_v, acc.at[sidx[s]], sem[s]).wait()

    for s in range(NSLOT):
        build(s, s)
        sstart(s)

    @pl.loop(0, NBW // NSLOT - 1)
    def _round(i):
        for s in range(NSLOT):
            swait(s)
            build(NSLOT * i + NSLOT + s, s)
            sstart(s)

    for s in range(NSLOT):
        swait(s)
    plsc.subcore_barrier()
    pltpu.sync_copy(
        acc.at[pl.ds(t * RPT, RPT)],
        deg_hbm.at[pl.ds(c * HALF + t * RPT, RPT)],
    )


@functools.partial(
    pl.kernel,
    out_type=(),
    mesh=_MESH,
    scratch_types=[
        pltpu.VMEM((EPW,), jnp.int32),
        pltpu.VMEM((EPW,), jnp.int32),
        pltpu.VMEM((K,), jnp.int32),
        pltpu.VMEM((K,), jnp.int32),
        pltpu.VMEM((K,), jnp.int32),
        pltpu.VMEM((K,), jnp.int32),
        pltpu.VMEM((K,), jnp.int32),
        pltpu.VMEM((K,), jnp.int32),
        pltpu.VMEM((K, D), jnp.float32),
        pltpu.VMEM((K, D), jnp.float32),
        pltpu.VMEM((K, D), jnp.float32),
        pltpu.SemaphoreType.DMA,
        pltpu.SemaphoreType.DMA,
        pltpu.SemaphoreType.DMA,
        pltpu.SemaphoreType.DMA,
        pltpu.SemaphoreType.DMA,
        pltpu.SemaphoreType.DMA,
    ],
)
def _sc_scatter(ei_hbm, hws_hbm, acc_ref,
                src_v, dst_v, gidx0, gidx1, gidx2, sidx0, sidx1, sidx2,
                rows0, rows1, rows2, gsem0, gsem1, gsem2,
                ssem0, ssem1, ssem2):
    gidx = (gidx0, gidx1, gidx2)
    sidx = (sidx0, sidx1, sidx2)
    rows = (rows0, rows1, rows2)
    gsem = (gsem0, gsem1, gsem2)
    ssem = (ssem0, ssem1, ssem2)
    w = lax.axis_index("c") * NS + lax.axis_index("s")
    _stage_edges(ei_hbm, 0, src_v, w, _pad_src)
    _stage_edges(ei_hbm, E, dst_v, w, _pad_dst)

    def gstart(b, s):
        for v in range(K // 16):
            gidx[s][pl.ds(v * 16, 16)] = src_v[pl.ds(b * K + v * 16, 16)]
            sidx[s][pl.ds(v * 16, 16)] = dst_v[pl.ds(b * K + v * 16, 16)]
        pltpu.async_copy(hws_hbm.at[gidx[s]], rows[s], gsem[s])

    def gwait(s):
        pltpu.make_async_copy(hws_hbm.at[gidx[s]], rows[s], gsem[s]).wait()

    def sstart(s):
        pltpu.async_copy(rows[s], acc_ref.at[sidx[s]], ssem[s], add=True)

    def swait(s):
        pltpu.make_async_copy(rows[s], acc_ref.at[sidx[s]], ssem[s]).wait()

    for s in range(NSLOT):
        gstart(s, s)

    @pl.loop(0, NBW // NSLOT - 1)
    def _round(i):
        for s in range(NSLOT):
            gwait(s)
            sstart(s)
        for s in range(NSLOT):
            swait(s)
            gstart(NSLOT * i + NSLOT + s, s)

    for s in range(NSLOT):
        gwait(s)
        sstart(s)
    for s in range(NSLOT):
        swait(s)


def _ln_relu(acc, hws, dinv_col, b, g, beta):
    t = (acc + hws) * dinv_col + b
    mu = jnp.mean(t, axis=-1, keepdims=True)
    var = jnp.mean((t - mu) ** 2, axis=-1, keepdims=True)
    tn = g * (t - mu) * lax.rsqrt(var + 1e-5) + beta
    return jnp.maximum(tn, 0.0)


def _enc_body(x_ref, wenc_ref, benc_ref, w1_ref, h0_ref, hw1_ref):
    h0 = jnp.maximum(
        jnp.dot(x_ref[...], wenc_ref[...], preferred_element_type=jnp.float32)
        + benc_ref[...], 0.0)
    h0_ref[...] = h0
    hw1_ref[...] = jnp.dot(
        h0, w1_ref[...], preferred_element_type=jnp.float32)


def _scale_body(deg_ref, hw1_ref, dinv_ref, hws1_ref):
    dinv = lax.rsqrt(deg_ref[...])
    dinv_ref[...] = dinv
    hws1_ref[...] = hw1_ref[...] * dinv[:, 0:1]


def _mid_body(acc_ref, hws_ref, hprev_ref, dinv_ref, b_ref, g_ref, beta_ref,
              w_ref, h_ref, hwsn_ref):
    dinv = dinv_ref[...][:, 0:1]
    h = _ln_relu(acc_ref[...], hws_ref[...], dinv,
                 b_ref[...], g_ref[...], beta_ref[...]) + hprev_ref[...]
    h_ref[...] = h
    hwsn_ref[...] = jnp.dot(
        h, w_ref[...], preferred_element_type=jnp.float32) * dinv


def _fin_body(acc_ref, hws_ref, hprev_ref, dinv_ref, b_ref, g_ref, beta_ref,
              w_ref, bout_ref, out_ref):
    dinv = dinv_ref[...][:, 0:1]
    h = _ln_relu(acc_ref[...], hws_ref[...], dinv,
                 b_ref[...], g_ref[...], beta_ref[...]) + hprev_ref[...]
    out_ref[...] = jnp.dot(
        h, w_ref[...], preferred_element_type=jnp.float32) + bout_ref[...]


_GRID = (NPAD // 320,)
_ROWS = pl.BlockSpec((320, D), lambda i: (i, 0))
_ROWS16 = pl.BlockSpec((320, DW), lambda i: (i, 0))
_WMAT = pl.BlockSpec((D, D), lambda i: (0, 0))
_VECB = pl.BlockSpec((1, D), lambda i: (0, 0))
_F32 = jnp.float32

_enc_call = pl.pallas_call(
    _enc_body,
    grid=_GRID,
    in_specs=[_ROWS, _WMAT, _VECB, _WMAT],
    out_specs=[_ROWS, _ROWS],
    out_shape=[
        jax.ShapeDtypeStruct((NPAD, D), _F32),
        jax.ShapeDtypeStruct((NPAD, D), _F32),
    ],
)

_scale_call = pl.pallas_call(
    _scale_body,
    grid=_GRID,
    in_specs=[_ROWS16, _ROWS],
    out_specs=[_ROWS16, _ROWS],
    out_shape=[
        jax.ShapeDtypeStruct((NPAD, DW), _F32),
        jax.ShapeDtypeStruct((NPAD, D), _F32),
    ],
)

_mid_call = pl.pallas_call(
    _mid_body,
    grid=_GRID,
    in_specs=[_ROWS, _ROWS, _ROWS, _ROWS16, _VECB, _VECB, _VECB, _WMAT],
    out_specs=[_ROWS, _ROWS],
    out_shape=[
        jax.ShapeDtypeStruct((NPAD, D), _F32),
        jax.ShapeDtypeStruct((NPAD, D), _F32),
    ],
)

_fin_call = pl.pallas_call(
    _fin_body,
    grid=_GRID,
    in_specs=[_ROWS, _ROWS, _ROWS, _ROWS16, _VECB, _VECB, _VECB, _WMAT, _VECB],
    out_specs=_ROWS,
    out_shape=jax.ShapeDtypeStruct((N, D), _F32),
)


def kernel(x, edge_index, W_enc, b_enc, W1, b1, g1, beta1,
           W2, b2, g2, beta2, W_out, b_out):
    ones_k = jnp.ones((RPT, DW), jnp.float32)
    acc1_ref = jax.new_ref(jnp.zeros((NPAD, D), jnp.float32))
    acc2_ref = jax.new_ref(jnp.zeros((NPAD, D), jnp.float32))

    ei_flat = edge_index.reshape(2 * E)
    deg = _sc_degree(ei_flat, ones_k)
    h0, hw1 = _enc_call(x, W_enc, b_enc.reshape(1, D), W1)
    dinv, hws1 = _scale_call(deg, hw1)
    _sc_scatter(ei_flat, hws1, acc1_ref)
    h1, hws2 = _mid_call(
        acc1_ref[...], hws1, h0, dinv, b1.reshape(1, D), g1.reshape(1, D),
        beta1.reshape(1, D), W2)
    _sc_scatter(ei_flat, hws2, acc2_ref)
    return _fin_call(
        acc2_ref[...], hws2, h1, dinv, b2.reshape(1, D), g2.reshape(1, D),
        beta2.reshape(1, D), W_out, b_out.reshape(1, D))
